# Initial kernel scaffold; baseline (speedup 1.0000x reference)
import jax
import jax.numpy as jnp
from jax.experimental import pallas as pl


def kernel(x, pos, W1, gamma1, beta1, W2, gamma2, beta2):
    raise NotImplementedError("write your pallas kernel here")


# speedup vs baseline: 10.5590x; 10.5590x over previous
import functools

import jax
import jax.numpy as jnp
from jax import lax
from jax.experimental import pallas as pl
from jax.experimental.pallas import tpu as pltpu
from jax.experimental.pallas import tpu_sc as plsc

B, N, D = 2, 2048, 768
K = 4
C1 = 256
TN = 256
NB = N // TN
HI = jax.lax.Precision.HIGHEST


def _k1_body(x_ref, pos_ref, w1_ref, h_ref, u_ref, v_ref, xx_ref):
    h = x_ref[0] + pos_ref[0]
    h_ref[0] = h
    w1a = w1_ref[:, :D]
    wd = w1_ref[:, D:] - w1a
    u_ref[0] = lax.dot_general(h, w1a, (((1,), (1,)), ((), ())),
                               precision=HI, preferred_element_type=jnp.float32)
    v_ref[0] = lax.dot_general(h, wd, (((1,), (1,)), ((), ())),
                               precision=HI, preferred_element_type=jnp.float32)
    xx_ref[0, 0] = jnp.sum(h * h, axis=1)


def _stage1(x, pos, W1):
    return pl.pallas_call(
        _k1_body,
        grid=(B, NB),
        in_specs=[
            pl.BlockSpec((1, TN, D), lambda b, nb: (b, nb, 0)),
            pl.BlockSpec((1, TN, D), lambda b, nb: (b, nb, 0)),
            pl.BlockSpec((C1, 2 * D), lambda b, nb: (0, 0)),
        ],
        out_specs=[
            pl.BlockSpec((1, TN, D), lambda b, nb: (b, nb, 0)),
            pl.BlockSpec((1, TN, C1), lambda b, nb: (b, nb, 0)),
            pl.BlockSpec((1, TN, C1), lambda b, nb: (b, nb, 0)),
            pl.BlockSpec((1, 1, TN), lambda b, nb: (b, 0, nb)),
        ],
        out_shape=[
            jax.ShapeDtypeStruct((B, N, D), jnp.float32),
            jax.ShapeDtypeStruct((B, N, C1), jnp.float32),
            jax.ShapeDtypeStruct((B, N, C1), jnp.float32),
            jax.ShapeDtypeStruct((B, 1, N), jnp.float32),
        ],
    )(x, pos, W1)


def _k2_body(hb_ref, hf_ref, xx_ref, idx_ref):
    b = pl.program_id(0)
    nb = pl.program_id(1)
    hb = hb_ref[0]
    hf = hf_ref[0]
    s = lax.dot_general(hb, hf, (((1,), (1,)), ((), ())),
                        preferred_element_type=jnp.float32)
    xxf = xx_ref[0, 0]
    xxb = xx_ref[0, 0, pl.ds(nb * TN, TN)]
    pd = (-xxb[:, None] + 2.0 * s) - xxf[None, :]
    iota = lax.broadcasted_iota(jnp.int32, (TN, N), 1)
    neg_inf = jnp.float32(float("-inf"))
    for k in range(K):
        m = jnp.max(pd, axis=1, keepdims=True)
        cand = jnp.where(pd == m, iota, N)
        sel = jnp.min(cand, axis=1)
        idx_ref[0, k, pl.ds(nb * TN, TN)] = sel + b * N
        if k < K - 1:
            pd = jnp.where(iota == sel[:, None], neg_inf, pd)


def _stage2(h, xx):
    return pl.pallas_call(
        _k2_body,
        grid=(B, NB),
        in_specs=[
            pl.BlockSpec((1, TN, D), lambda b, nb: (b, nb, 0)),
            pl.BlockSpec((1, N, D), lambda b, nb: (b, 0, 0)),
            pl.BlockSpec((1, 1, N), lambda b, nb: (b, 0, 0)),
        ],
        out_specs=pl.BlockSpec((1, K, N), lambda b, nb: (b, 0, 0)),
        out_shape=jax.ShapeDtypeStruct((B, K, N), jnp.int32),
    )(h, h, xx)


NG = B * K * N
CH = 128


def _sc_gather(table, idx):
    info = plsc.get_sparse_core_info()
    nw = info.num_cores * info.num_subcores
    per_w = NG // nw
    mesh = plsc.VectorSubcoreMesh(core_axis_name="c", subcore_axis_name="s")

    @functools.partial(
        pl.kernel,
        mesh=mesh,
        out_type=jax.ShapeDtypeStruct((NG, C1), jnp.float32),
        scratch_types=[
            pltpu.VMEM((CH,), jnp.int32),
            pltpu.VMEM((CH, C1), jnp.float32),
            pltpu.SemaphoreType.DMA,
        ],
    )
    def gather_k(table_hbm, idx_hbm, out_hbm, idx_v, rows_v, sem):
        wid = lax.axis_index("s") * info.num_cores + lax.axis_index("c")
        base = wid * per_w
        for c in range(per_w // CH):
            off = base + c * CH
            pltpu.sync_copy(idx_hbm.at[pl.ds(off, CH)], idx_v)
            pltpu.async_copy(table_hbm.at[idx_v], rows_v, sem).wait()
            pltpu.sync_copy(rows_v, out_hbm.at[pl.ds(off, CH)])

    return gather_k(table, idx)


def _k4a_body(g_ref, v_ref, ymax_ref, s1_ref, q1_ref):
    first = (pl.program_id(0) == 0) & (pl.program_id(1) == 0)

    @pl.when(first)
    def _():
        s1_ref[...] = jnp.zeros_like(s1_ref)
        q1_ref[...] = jnp.zeros_like(q1_ref)

    y = g_ref[0] + v_ref[0][None]
    ymax_ref[0] = jnp.max(y, axis=0)
    s1_ref[0] += jnp.sum(y, axis=(0, 1))
    q1_ref[0] += jnp.sum(y * y, axis=(0, 1))


def _stage4a(g4, v):
    return pl.pallas_call(
        _k4a_body,
        grid=(B, NB),
        in_specs=[
            pl.BlockSpec((1, K, TN, C1), lambda b, nb: (b, 0, nb, 0)),
            pl.BlockSpec((1, TN, C1), lambda b, nb: (b, nb, 0)),
        ],
        out_specs=[
            pl.BlockSpec((1, TN, C1), lambda b, nb: (b, nb, 0)),
            pl.BlockSpec((1, C1), lambda b, nb: (0, 0)),
            pl.BlockSpec((1, C1), lambda b, nb: (0, 0)),
        ],
        out_shape=[
            jax.ShapeDtypeStruct((B, N, C1), jnp.float32),
            jax.ShapeDtypeStruct((1, C1), jnp.float32),
            jax.ShapeDtypeStruct((1, C1), jnp.float32),
        ],
    )(g4, v)


def _k4b_body(ym_ref, s1_ref, q1_ref, g1_ref, b1_ref, w2_ref,
              z_ref, s2_ref, q2_ref):
    first = (pl.program_id(0) == 0) & (pl.program_id(1) == 0)

    @pl.when(first)
    def _():
        s2_ref[...] = jnp.zeros_like(s2_ref)
        q2_ref[...] = jnp.zeros_like(q2_ref)

    cnt = jnp.float32(B * N * K)
    mean = s1_ref[0] / cnt
    var = q1_ref[0] / cnt - mean * mean
    scale = lax.rsqrt(var + 1e-5) * g1_ref[0]
    shift = b1_ref[0] - mean * scale
    t = ym_ref[0] * scale[None, :] + shift[None, :]
    t = jnp.where(t >= 0, t, 0.2 * t)
    z = lax.dot_general(t, w2_ref[...], (((1,), (1,)), ((), ())),
                        precision=HI, preferred_element_type=jnp.float32)
    z_ref[0] = z
    s2_ref[0] += jnp.sum(z, axis=0)
    q2_ref[0] += jnp.sum(z * z, axis=0)


def _stage4b(ymax, s1, q1, gamma1, beta1, W2):
    return pl.pallas_call(
        _k4b_body,
        grid=(B, NB),
        in_specs=[
            pl.BlockSpec((1, TN, C1), lambda b, nb: (b, nb, 0)),
            pl.BlockSpec((1, C1), lambda b, nb: (0, 0)),
            pl.BlockSpec((1, C1), lambda b, nb: (0, 0)),
            pl.BlockSpec((1, C1), lambda b, nb: (0, 0)),
            pl.BlockSpec((1, C1), lambda b, nb: (0, 0)),
            pl.BlockSpec((D, C1), lambda b, nb: (0, 0)),
        ],
        out_specs=[
            pl.BlockSpec((1, TN, D), lambda b, nb: (b, nb, 0)),
            pl.BlockSpec((1, D), lambda b, nb: (0, 0)),
            pl.BlockSpec((1, D), lambda b, nb: (0, 0)),
        ],
        out_shape=[
            jax.ShapeDtypeStruct((B, N, D), jnp.float32),
            jax.ShapeDtypeStruct((1, D), jnp.float32),
            jax.ShapeDtypeStruct((1, D), jnp.float32),
        ],
    )(ymax, s1, q1, gamma1, beta1, W2)


def _k4c_body(z_ref, s2_ref, q2_ref, g2_ref, b2_ref, out_ref):
    cnt = jnp.float32(B * N)
    mean = s2_ref[0] / cnt
    var = q2_ref[0] / cnt - mean * mean
    scale = lax.rsqrt(var + 1e-5) * g2_ref[0]
    shift = b2_ref[0] - mean * scale
    o = z_ref[0] * scale[None, :] + shift[None, :]
    out_ref[0] = jnp.where(o >= 0, o, 0.2 * o)


def _stage4c(z, s2, q2, gamma2, beta2):
    return pl.pallas_call(
        _k4c_body,
        grid=(B, NB),
        in_specs=[
            pl.BlockSpec((1, TN, D), lambda b, nb: (b, nb, 0)),
            pl.BlockSpec((1, D), lambda b, nb: (0, 0)),
            pl.BlockSpec((1, D), lambda b, nb: (0, 0)),
            pl.BlockSpec((1, D), lambda b, nb: (0, 0)),
            pl.BlockSpec((1, D), lambda b, nb: (0, 0)),
        ],
        out_specs=pl.BlockSpec((1, TN, D), lambda b, nb: (b, nb, 0)),
        out_shape=jax.ShapeDtypeStruct((B, N, D), jnp.float32),
    )(z, s2, q2, gamma2, beta2)


def kernel(x, pos, W1, gamma1, beta1, W2, gamma2, beta2):
    h, u, v, xx = _stage1(x, pos, W1)
    idx = _stage2(h, xx)
    g = _sc_gather(u.reshape(B * N, C1), idx.reshape(NG))
    g4 = g.reshape(B, K, N, C1)
    ymax, s1, q1 = _stage4a(g4, v)
    z, s2, q2 = _stage4b(ymax, s1, q1,
                         gamma1.reshape(1, C1), beta1.reshape(1, C1), W2)
    return _stage4c(z, s2, q2, gamma2.reshape(1, D), beta2.reshape(1, D))


# speedup vs baseline: 12.6731x; 1.2002x over previous
import functools

import jax
import jax.numpy as jnp
from jax import lax
from jax.experimental import pallas as pl
from jax.experimental.pallas import tpu as pltpu
from jax.experimental.pallas import tpu_sc as plsc

B, N, D = 2, 2048, 768
K = 4
C1 = 256
TN = 256
NB = N // TN
HI = jax.lax.Precision.HIGHEST


def _k1_body(x_ref, pos_ref, w1_ref, h_ref, u_ref, v_ref, xx_ref):
    h = x_ref[0] + pos_ref[0]
    h_ref[0] = h
    w1a = w1_ref[:, :D]
    wd = w1_ref[:, D:] - w1a
    u_ref[0] = lax.dot_general(h, w1a, (((1,), (1,)), ((), ())),
                               preferred_element_type=jnp.float32)
    v_ref[0] = lax.dot_general(h, wd, (((1,), (1,)), ((), ())),
                               preferred_element_type=jnp.float32)
    xx_ref[0, 0] = jnp.sum(h * h, axis=1)


def _stage1(x, pos, W1):
    return pl.pallas_call(
        _k1_body,
        grid=(B, NB),
        in_specs=[
            pl.BlockSpec((1, TN, D), lambda b, nb: (b, nb, 0)),
            pl.BlockSpec((1, TN, D), lambda b, nb: (b, nb, 0)),
            pl.BlockSpec((C1, 2 * D), lambda b, nb: (0, 0)),
        ],
        out_specs=[
            pl.BlockSpec((1, TN, D), lambda b, nb: (b, nb, 0)),
            pl.BlockSpec((1, TN, C1), lambda b, nb: (b, nb, 0)),
            pl.BlockSpec((1, TN, C1), lambda b, nb: (b, nb, 0)),
            pl.BlockSpec((1, 1, TN), lambda b, nb: (b, 0, nb)),
        ],
        out_shape=[
            jax.ShapeDtypeStruct((B, N, D), jnp.float32),
            jax.ShapeDtypeStruct((B, N, C1), jnp.float32),
            jax.ShapeDtypeStruct((B, N, C1), jnp.float32),
            jax.ShapeDtypeStruct((B, 1, N), jnp.float32),
        ],
    )(x, pos, W1)


T2 = 512
NB2 = N // T2


def _k2_body(hb_ref, hf_ref, xx_ref, idx_ref):
    b = pl.program_id(0)
    nb = pl.program_id(1)
    hb = hb_ref[0]
    hf = hf_ref[0]
    s = lax.dot_general(hf, hb, (((1,), (1,)), ((), ())),
                        preferred_element_type=jnp.float32)
    xxf = xx_ref[0, 0]
    xxb = xx_ref[0, 0, pl.ds(nb * T2, T2)]
    pd = (-xxb[None, :] + 2.0 * s) - xxf[:, None]
    iota = lax.broadcasted_iota(jnp.int32, (N, T2), 0)
    neg_inf = jnp.float32(float("-inf"))
    for k in range(K):
        m = jnp.max(pd, axis=0, keepdims=True)
        cand = jnp.where(pd == m, iota, N)
        sel = jnp.min(cand, axis=0)
        idx_ref[0, k, pl.ds(nb * T2, T2)] = sel + b * N
        if k < K - 1:
            pd = jnp.where(iota == sel[None, :], neg_inf, pd)


def _stage2(h, xx):
    return pl.pallas_call(
        _k2_body,
        grid=(B, NB2),
        in_specs=[
            pl.BlockSpec((1, T2, D), lambda b, nb: (b, nb, 0)),
            pl.BlockSpec((1, N, D), lambda b, nb: (b, 0, 0)),
            pl.BlockSpec((1, 1, N), lambda b, nb: (b, 0, 0)),
        ],
        out_specs=pl.BlockSpec((1, K, N), lambda b, nb: (b, 0, 0)),
        out_shape=jax.ShapeDtypeStruct((B, K, N), jnp.int32),
    )(h, h, xx)


NG = B * K * N
CH = 128


def _sc_gather(table, idx):
    info = plsc.get_sparse_core_info()
    nw = info.num_cores * info.num_subcores
    per_w = NG // nw
    mesh = plsc.VectorSubcoreMesh(core_axis_name="c", subcore_axis_name="s")

    @functools.partial(
        pl.kernel,
        mesh=mesh,
        out_type=jax.ShapeDtypeStruct((NG, C1), jnp.float32),
        scratch_types=[
            pltpu.VMEM((CH,), jnp.int32),
            pltpu.VMEM((CH, C1), jnp.float32),
            pltpu.SemaphoreType.DMA,
        ],
    )
    def gather_k(table_hbm, idx_hbm, out_hbm, idx_v, rows_v, sem):
        wid = lax.axis_index("s") * info.num_cores + lax.axis_index("c")
        base = wid * per_w
        for c in range(per_w // CH):
            off = base + c * CH
            pltpu.sync_copy(idx_hbm.at[pl.ds(off, CH)], idx_v)
            pltpu.async_copy(table_hbm.at[idx_v], rows_v, sem).wait()
            pltpu.sync_copy(rows_v, out_hbm.at[pl.ds(off, CH)])

    return gather_k(table, idx)


def _k4_body(g_ref, v_ref, g1_ref, b1_ref, w2_ref, g2_ref, b2_ref,
             out_ref, ym_scr, z_scr, s1_scr, q1_scr, s2_scr, q2_scr):
    p = pl.program_id(0)
    b = pl.program_id(1)
    nb = pl.program_id(2)
    first = (b == 0) & (nb == 0)

    @pl.when((p == 0) & first)
    def _():
        s1_scr[...] = jnp.zeros_like(s1_scr)
        q1_scr[...] = jnp.zeros_like(q1_scr)
        s2_scr[...] = jnp.zeros_like(s2_scr)
        q2_scr[...] = jnp.zeros_like(q2_scr)

    @pl.when(p == 0)
    def _():
        y = g_ref[0] + v_ref[0][None]
        ym_scr[b, pl.ds(nb * TN, TN), :] = jnp.max(y, axis=0)
        s1_scr[0, :] += jnp.sum(y, axis=(0, 1))
        q1_scr[0, :] += jnp.sum(y * y, axis=(0, 1))

    @pl.when(p == 1)
    def _():
        cnt = jnp.float32(B * N * K)
        mean = s1_scr[0, :] / cnt
        var = q1_scr[0, :] / cnt - mean * mean
        scale = lax.rsqrt(var + 1e-5) * g1_ref[0]
        shift = b1_ref[0] - mean * scale
        t = ym_scr[b, pl.ds(nb * TN, TN), :] * scale[None, :] + shift[None, :]
        t = jnp.where(t >= 0, t, 0.2 * t)
        z = lax.dot_general(t, w2_ref[...], (((1,), (1,)), ((), ())),
                            precision=HI, preferred_element_type=jnp.float32)
        z_scr[b, pl.ds(nb * TN, TN), :] = z
        s2_scr[0, :] += jnp.sum(z, axis=0)
        q2_scr[0, :] += jnp.sum(z * z, axis=0)

    @pl.when(p == 2)
    def _():
        cnt = jnp.float32(B * N)
        mean = s2_scr[0, :] / cnt
        var = q2_scr[0, :] / cnt - mean * mean
        scale = lax.rsqrt(var + 1e-5) * g2_ref[0]
        shift = b2_ref[0] - mean * scale
        o = z_scr[b, pl.ds(nb * TN, TN), :] * scale[None, :] + shift[None, :]
        out_ref[0] = jnp.where(o >= 0, o, 0.2 * o)


def _stage4(g4, v, gamma1, beta1, W2, gamma2, beta2):
    def ph0(p, b, nb):
        return (b * (p == 0), 0, nb * (p == 0), 0)

    def ph0v(p, b, nb):
        return (b * (p == 0), nb * (p == 0), 0)

    return pl.pallas_call(
        _k4_body,
        grid=(3, B, NB),
        in_specs=[
            pl.BlockSpec((1, K, TN, C1), ph0),
            pl.BlockSpec((1, TN, C1), ph0v),
            pl.BlockSpec((1, C1), lambda p, b, nb: (0, 0)),
            pl.BlockSpec((1, C1), lambda p, b, nb: (0, 0)),
            pl.BlockSpec((D, C1), lambda p, b, nb: (0, 0)),
            pl.BlockSpec((1, D), lambda p, b, nb: (0, 0)),
            pl.BlockSpec((1, D), lambda p, b, nb: (0, 0)),
        ],
        out_specs=pl.BlockSpec(
            (1, TN, D),
            lambda p, b, nb: (b * (p == 2), nb * (p == 2), 0)),
        out_shape=jax.ShapeDtypeStruct((B, N, D), jnp.float32),
        scratch_shapes=[
            pltpu.VMEM((B, N, C1), jnp.float32),
            pltpu.VMEM((B, N, D), jnp.float32),
            pltpu.VMEM((1, C1), jnp.float32),
            pltpu.VMEM((1, C1), jnp.float32),
            pltpu.VMEM((1, D), jnp.float32),
            pltpu.VMEM((1, D), jnp.float32),
        ],
    )(g4, v, gamma1, beta1, W2, gamma2, beta2)


def kernel(x, pos, W1, gamma1, beta1, W2, gamma2, beta2):
    h, u, v, xx = _stage1(x, pos, W1)
    idx = _stage2(h, xx)
    g = _sc_gather(u.reshape(B * N, C1), idx.reshape(NG))
    g4 = g.reshape(B, K, N, C1)
    return _stage4(g4, v, gamma1.reshape(1, C1), beta1.reshape(1, C1),
                   W2, gamma2.reshape(1, D), beta2.reshape(1, D))


# speedup vs baseline: 13.6252x; 1.0751x over previous
import functools

import jax
import jax.numpy as jnp
from jax import lax
from jax.experimental import pallas as pl
from jax.experimental.pallas import tpu as pltpu
from jax.experimental.pallas import tpu_sc as plsc

B, N, D = 2, 2048, 768
K = 4
C1 = 256
TN = 256
NB = N // TN
HI = jax.lax.Precision.HIGHEST


def _k1_body(x_ref, pos_ref, w1_ref, h_ref, u_ref, v_ref, xx_ref):
    h = x_ref[0] + pos_ref[0]
    h_ref[0] = h
    w1a = w1_ref[:, :D]
    wd = w1_ref[:, D:] - w1a
    u_ref[0] = lax.dot_general(h, w1a, (((1,), (1,)), ((), ())),
                               preferred_element_type=jnp.float32)
    v_ref[0] = lax.dot_general(h, wd, (((1,), (1,)), ((), ())),
                               preferred_element_type=jnp.float32)
    xx_ref[0, 0] = jnp.sum(h * h, axis=1)


def _stage1(x, pos, W1):
    return pl.pallas_call(
        _k1_body,
        grid=(B, NB),
        in_specs=[
            pl.BlockSpec((1, TN, D), lambda b, nb: (b, nb, 0)),
            pl.BlockSpec((1, TN, D), lambda b, nb: (b, nb, 0)),
            pl.BlockSpec((C1, 2 * D), lambda b, nb: (0, 0)),
        ],
        out_specs=[
            pl.BlockSpec((1, TN, D), lambda b, nb: (b, nb, 0)),
            pl.BlockSpec((1, TN, C1), lambda b, nb: (b, nb, 0)),
            pl.BlockSpec((1, TN, C1), lambda b, nb: (b, nb, 0)),
            pl.BlockSpec((1, 1, TN), lambda b, nb: (b, 0, nb)),
        ],
        out_shape=[
            jax.ShapeDtypeStruct((B, N, D), jnp.float32),
            jax.ShapeDtypeStruct((B, N, C1), jnp.float32),
            jax.ShapeDtypeStruct((B, N, C1), jnp.float32),
            jax.ShapeDtypeStruct((B, 1, N), jnp.float32),
        ],
    )(x, pos, W1)


T2 = 512
NB2 = N // T2


def _k2_body(hb_ref, hf_ref, xx_ref, idx_ref):
    b = pl.program_id(0)
    nb = pl.program_id(1)
    hb = hb_ref[0]
    hf = hf_ref[0]
    s = lax.dot_general(hf, hb, (((1,), (1,)), ((), ())),
                        preferred_element_type=jnp.float32)
    xxf = xx_ref[0, 0]
    xxb = xx_ref[0, 0, pl.ds(nb * T2, T2)]
    pd = (-xxb[None, :] + 2.0 * s) - xxf[:, None]
    iota = lax.broadcasted_iota(jnp.int32, (N, T2), 0)
    neg_inf = jnp.float32(float("-inf"))
    colg = lax.broadcasted_iota(jnp.int32, (N, T2), 1) + nb * T2
    pd = jnp.where(iota == colg, neg_inf, pd)
    for k in range(K - 1):
        m = jnp.max(pd, axis=0, keepdims=True)
        cand = jnp.where(pd == m, iota, N)
        sel = jnp.min(cand, axis=0)
        idx_ref[0, k, pl.ds(nb * T2, T2)] = sel + b * N
        if k < K - 2:
            pd = jnp.where(iota == sel[None, :], neg_inf, pd)


def _stage2(h, xx):
    return pl.pallas_call(
        _k2_body,
        grid=(B, NB2),
        in_specs=[
            pl.BlockSpec((1, T2, D), lambda b, nb: (b, nb, 0)),
            pl.BlockSpec((1, N, D), lambda b, nb: (b, 0, 0)),
            pl.BlockSpec((1, 1, N), lambda b, nb: (b, 0, 0)),
        ],
        out_specs=pl.BlockSpec((1, K - 1, N), lambda b, nb: (b, 0, 0)),
        out_shape=jax.ShapeDtypeStruct((B, K - 1, N), jnp.int32),
    )(h, h, xx)


NG = B * (K - 1) * N
CH = 128
NCH = 3


def _sc_gather(table, idx):
    info = plsc.get_sparse_core_info()
    nw = info.num_cores * info.num_subcores
    per_w = NG // nw
    mesh = plsc.VectorSubcoreMesh(core_axis_name="c", subcore_axis_name="s")

    @functools.partial(
        pl.kernel,
        mesh=mesh,
        out_type=jax.ShapeDtypeStruct((NG, C1), jnp.float32),
        scratch_types=[
            [pltpu.VMEM((CH,), jnp.int32)] * NCH,
            [pltpu.VMEM((CH, C1), jnp.float32)] * NCH,
            [pltpu.SemaphoreType.DMA] * NCH,
            [pltpu.SemaphoreType.DMA] * NCH,
        ],
    )
    def gather_k(table_hbm, idx_hbm, out_hbm, idx_vs, rows_vs, sgs, sws):
        wid = lax.axis_index("s") * info.num_cores + lax.axis_index("c")
        base = wid * per_w
        gathers = []
        for c in range(NCH):
            pltpu.sync_copy(idx_hbm.at[pl.ds(base + c * CH, CH)], idx_vs[c])
            gathers.append(
                pltpu.async_copy(table_hbm.at[idx_vs[c]], rows_vs[c], sgs[c]))
        writes = []
        for c in range(NCH):
            gathers[c].wait()
            writes.append(
                pltpu.async_copy(rows_vs[c],
                                 out_hbm.at[pl.ds(base + c * CH, CH)], sws[c]))
        for c in range(NCH):
            writes[c].wait()

    return gather_k(table, idx)


def _k4_body(g_ref, u_ref, v_ref, g1_ref, b1_ref, w2_ref, g2_ref, b2_ref,
             out_ref, ym_scr, z_scr, s1_scr, q1_scr, s2_scr, q2_scr):
    p = pl.program_id(0)
    b = pl.program_id(1)
    nb = pl.program_id(2)
    first = (b == 0) & (nb == 0)

    @pl.when((p == 0) & first)
    def _():
        s1_scr[...] = jnp.zeros_like(s1_scr)
        q1_scr[...] = jnp.zeros_like(q1_scr)
        s2_scr[...] = jnp.zeros_like(s2_scr)
        q2_scr[...] = jnp.zeros_like(q2_scr)

    @pl.when(p == 0)
    def _():
        y = g_ref[0] + v_ref[0][None]
        y0 = u_ref[0] + v_ref[0]
        ym_scr[b, pl.ds(nb * TN, TN), :] = jnp.maximum(jnp.max(y, axis=0), y0)
        s1_scr[0, :] += jnp.sum(y, axis=(0, 1)) + jnp.sum(y0, axis=0)
        q1_scr[0, :] += jnp.sum(y * y, axis=(0, 1)) + jnp.sum(y0 * y0, axis=0)

    @pl.when(p == 1)
    def _():
        cnt = jnp.float32(B * N * K)
        mean = s1_scr[0, :] / cnt
        var = q1_scr[0, :] / cnt - mean * mean
        scale = lax.rsqrt(var + 1e-5) * g1_ref[0]
        shift = b1_ref[0] - mean * scale
        t = ym_scr[b, pl.ds(nb * TN, TN), :] * scale[None, :] + shift[None, :]
        t = jnp.where(t >= 0, t, 0.2 * t)
        z = lax.dot_general(t, w2_ref[...], (((1,), (1,)), ((), ())),
                            precision=HI, preferred_element_type=jnp.float32)
        z_scr[b, pl.ds(nb * TN, TN), :] = z
        s2_scr[0, :] += jnp.sum(z, axis=0)
        q2_scr[0, :] += jnp.sum(z * z, axis=0)

    @pl.when(p == 2)
    def _():
        cnt = jnp.float32(B * N)
        mean = s2_scr[0, :] / cnt
        var = q2_scr[0, :] / cnt - mean * mean
        scale = lax.rsqrt(var + 1e-5) * g2_ref[0]
        shift = b2_ref[0] - mean * scale
        o = z_scr[b, pl.ds(nb * TN, TN), :] * scale[None, :] + shift[None, :]
        out_ref[0] = jnp.where(o >= 0, o, 0.2 * o)


def _stage4(g4, u, v, gamma1, beta1, W2, gamma2, beta2):
    def ph0(p, b, nb):
        return (b * (p == 0), 0, nb * (p == 0), 0)

    def ph0v(p, b, nb):
        return (b * (p == 0), nb * (p == 0), 0)

    return pl.pallas_call(
        _k4_body,
        grid=(3, B, NB),
        in_specs=[
            pl.BlockSpec((1, K - 1, TN, C1), ph0),
            pl.BlockSpec((1, TN, C1), ph0v),
            pl.BlockSpec((1, TN, C1), ph0v),
            pl.BlockSpec((1, C1), lambda p, b, nb: (0, 0)),
            pl.BlockSpec((1, C1), lambda p, b, nb: (0, 0)),
            pl.BlockSpec((D, C1), lambda p, b, nb: (0, 0)),
            pl.BlockSpec((1, D), lambda p, b, nb: (0, 0)),
            pl.BlockSpec((1, D), lambda p, b, nb: (0, 0)),
        ],
        out_specs=pl.BlockSpec(
            (1, TN, D),
            lambda p, b, nb: (b * (p == 2), nb * (p == 2), 0)),
        out_shape=jax.ShapeDtypeStruct((B, N, D), jnp.float32),
        scratch_shapes=[
            pltpu.VMEM((B, N, C1), jnp.float32),
            pltpu.VMEM((B, N, D), jnp.float32),
            pltpu.VMEM((1, C1), jnp.float32),
            pltpu.VMEM((1, C1), jnp.float32),
            pltpu.VMEM((1, D), jnp.float32),
            pltpu.VMEM((1, D), jnp.float32),
        ],
    )(g4, u, v, gamma1, beta1, W2, gamma2, beta2)


def kernel(x, pos, W1, gamma1, beta1, W2, gamma2, beta2):
    h, u, v, xx = _stage1(x, pos, W1)
    idx = _stage2(h, xx)
    g = _sc_gather(u.reshape(B * N, C1), idx.reshape(NG))
    g4 = g.reshape(B, K - 1, N, C1)
    return _stage4(g4, u, v, gamma1.reshape(1, C1), beta1.reshape(1, C1),
                   W2, gamma2.reshape(1, D), beta2.reshape(1, D))


# speedup vs baseline: 14.4208x; 1.0584x over previous
import functools

import jax
import jax.numpy as jnp
from jax import lax
from jax.experimental import pallas as pl
from jax.experimental.pallas import tpu as pltpu
from jax.experimental.pallas import tpu_sc as plsc

B, N, D = 2, 2048, 768
K = 4
C1 = 256
TN = 256
NB = N // TN
HI = jax.lax.Precision.HIGHEST


T2 = 512
NB2 = N // T2


def _kA_body(x_ref, pos_ref, w1_ref, u_ref, v_ref, idx_ref, h_scr, xx_scr):
    p = pl.program_id(0)
    b = pl.program_id(1)
    nb = pl.program_id(2)

    @pl.when(p == 0)
    def _():
        h = x_ref[0] + pos_ref[0]
        h_scr[b, pl.ds(nb * T2, T2), :] = h
        w1a = w1_ref[:, :D]
        wd = w1_ref[:, D:] - w1a
        u_ref[0] = lax.dot_general(h, w1a, (((1,), (1,)), ((), ())),
                                   preferred_element_type=jnp.float32)
        v_ref[0] = lax.dot_general(h, wd, (((1,), (1,)), ((), ())),
                                   preferred_element_type=jnp.float32)
        xx_scr[b, 0, pl.ds(nb * T2, T2)] = jnp.sum(h * h, axis=1)

    @pl.when(p == 1)
    def _():
        hb = h_scr[b, pl.ds(nb * T2, T2), :]
        hf = h_scr[b]
        s = lax.dot_general(hf, hb, (((1,), (1,)), ((), ())),
                            preferred_element_type=jnp.float32)
        xxf = xx_scr[b, 0, :]
        xxb = xx_scr[b, 0, pl.ds(nb * T2, T2)]
        pd = (-xxb[None, :] + 2.0 * s) - xxf[:, None]
        iota = lax.broadcasted_iota(jnp.int32, (N, T2), 0)
        neg_inf = jnp.float32(float("-inf"))
        colg = lax.broadcasted_iota(jnp.int32, (N, T2), 1) + nb * T2
        pd = jnp.where(iota == colg, neg_inf, pd)
        for k in range(K - 1):
            m = jnp.max(pd, axis=0, keepdims=True)
            cand = jnp.where(pd == m, iota, N)
            sel = jnp.min(cand, axis=0)
            idx_ref[0, k, pl.ds(nb * T2, T2)] = sel + b * N
            if k < K - 2:
                pd = jnp.where(iota == sel[None, :], neg_inf, pd)


def _stageA(x, pos, W1):
    def in_map(p, b, nb):
        return (jnp.where(p == 0, b, B - 1),
                jnp.where(p == 0, nb, NB2 - 1), 0)

    def uv_map(p, b, nb):
        return (jnp.where(p == 0, b, B - 1),
                jnp.where(p == 0, nb, NB2 - 1), 0)

    def idx_map(p, b, nb):
        return (jnp.where(p == 0, 0, b), 0, 0)

    return pl.pallas_call(
        _kA_body,
        grid=(2, B, NB2),
        in_specs=[
            pl.BlockSpec((1, T2, D), in_map),
            pl.BlockSpec((1, T2, D), in_map),
            pl.BlockSpec((C1, 2 * D), lambda p, b, nb: (0, 0)),
        ],
        out_specs=[
            pl.BlockSpec((1, T2, C1), uv_map),
            pl.BlockSpec((1, T2, C1), uv_map),
            pl.BlockSpec((1, K - 1, N), idx_map),
        ],
        out_shape=[
            jax.ShapeDtypeStruct((B, N, C1), jnp.float32),
            jax.ShapeDtypeStruct((B, N, C1), jnp.float32),
            jax.ShapeDtypeStruct((B, K - 1, N), jnp.int32),
        ],
        scratch_shapes=[
            pltpu.VMEM((B, N, D), jnp.float32),
            pltpu.VMEM((B, 1, N), jnp.float32),
        ],
    )(x, pos, W1)


NG = B * (K - 1) * N


def _sc_gather(table, idx):
    info = plsc.get_sparse_core_info()
    nw = info.num_cores * info.num_subcores
    per_w = NG // nw
    CH = 128
    NCH = per_w // CH
    mesh = plsc.VectorSubcoreMesh(core_axis_name="c", subcore_axis_name="s")

    @functools.partial(
        pl.kernel,
        mesh=mesh,
        out_type=jax.ShapeDtypeStruct((NG, C1), jnp.float32),
        scratch_types=[
            [pltpu.VMEM((CH,), jnp.int32)] * NCH,
            [pltpu.VMEM((CH, C1), jnp.float32)] * NCH,
            [pltpu.SemaphoreType.DMA] * NCH,
            [pltpu.SemaphoreType.DMA] * NCH,
        ],
    )
    def gather_k(table_hbm, idx_hbm, out_hbm, idx_vs, rows_vs, sgs, sws):
        wid = lax.axis_index("s") * info.num_cores + lax.axis_index("c")
        base = wid * per_w
        gathers = []
        for c in range(NCH):
            pltpu.sync_copy(idx_hbm.at[pl.ds(base + c * CH, CH)], idx_vs[c])
            gathers.append(
                pltpu.async_copy(table_hbm.at[idx_vs[c]], rows_vs[c], sgs[c]))
        writes = []
        for c in range(NCH):
            gathers[c].wait()
            writes.append(
                pltpu.async_copy(rows_vs[c],
                                 out_hbm.at[pl.ds(base + c * CH, CH)], sws[c]))
        for c in range(NCH):
            writes[c].wait()

    return gather_k(table, idx)


def _k4_body(g_ref, u_ref, v_ref, g1_ref, b1_ref, w2_ref, g2_ref, b2_ref,
             out_ref, ym_scr, z_scr, s1_scr, q1_scr, s2_scr, q2_scr):
    p = pl.program_id(0)
    b = pl.program_id(1)
    nb = pl.program_id(2)
    first = (b == 0) & (nb == 0)

    @pl.when((p == 0) & first)
    def _():
        s1_scr[...] = jnp.zeros_like(s1_scr)
        q1_scr[...] = jnp.zeros_like(q1_scr)
        s2_scr[...] = jnp.zeros_like(s2_scr)
        q2_scr[...] = jnp.zeros_like(q2_scr)

    @pl.when(p == 0)
    def _():
        y = g_ref[0] + v_ref[0][None]
        y0 = u_ref[0] + v_ref[0]
        ym_scr[b, pl.ds(nb * TN, TN), :] = jnp.maximum(jnp.max(y, axis=0), y0)
        s1_scr[0, :] += jnp.sum(y, axis=(0, 1)) + jnp.sum(y0, axis=0)
        q1_scr[0, :] += jnp.sum(y * y, axis=(0, 1)) + jnp.sum(y0 * y0, axis=0)

    @pl.when(p == 1)
    def _():
        cnt = jnp.float32(B * N * K)
        mean = s1_scr[0, :] / cnt
        var = q1_scr[0, :] / cnt - mean * mean
        scale = lax.rsqrt(var + 1e-5) * g1_ref[0]
        shift = b1_ref[0] - mean * scale
        t = ym_scr[b, pl.ds(nb * TN, TN), :] * scale[None, :] + shift[None, :]
        t = jnp.where(t >= 0, t, 0.2 * t)
        z = lax.dot_general(t, w2_ref[...], (((1,), (1,)), ((), ())),
                            precision=HI, preferred_element_type=jnp.float32)
        z_scr[b, pl.ds(nb * TN, TN), :] = z
        s2_scr[0, :] += jnp.sum(z, axis=0)
        q2_scr[0, :] += jnp.sum(z * z, axis=0)

    @pl.when(p == 2)
    def _():
        cnt = jnp.float32(B * N)
        mean = s2_scr[0, :] / cnt
        var = q2_scr[0, :] / cnt - mean * mean
        scale = lax.rsqrt(var + 1e-5) * g2_ref[0]
        shift = b2_ref[0] - mean * scale
        o = z_scr[b, pl.ds(nb * TN, TN), :] * scale[None, :] + shift[None, :]
        out_ref[0] = jnp.where(o >= 0, o, 0.2 * o)


def _stage4(g4, u, v, gamma1, beta1, W2, gamma2, beta2):
    def ph0(p, b, nb):
        return (b * (p == 0), 0, nb * (p == 0), 0)

    def ph0v(p, b, nb):
        return (b * (p == 0), nb * (p == 0), 0)

    return pl.pallas_call(
        _k4_body,
        grid=(3, B, NB),
        in_specs=[
            pl.BlockSpec((1, K - 1, TN, C1), ph0),
            pl.BlockSpec((1, TN, C1), ph0v),
            pl.BlockSpec((1, TN, C1), ph0v),
            pl.BlockSpec((1, C1), lambda p, b, nb: (0, 0)),
            pl.BlockSpec((1, C1), lambda p, b, nb: (0, 0)),
            pl.BlockSpec((D, C1), lambda p, b, nb: (0, 0)),
            pl.BlockSpec((1, D), lambda p, b, nb: (0, 0)),
            pl.BlockSpec((1, D), lambda p, b, nb: (0, 0)),
        ],
        out_specs=pl.BlockSpec(
            (1, TN, D),
            lambda p, b, nb: (b * (p == 2), nb * (p == 2), 0)),
        out_shape=jax.ShapeDtypeStruct((B, N, D), jnp.float32),
        scratch_shapes=[
            pltpu.VMEM((B, N, C1), jnp.float32),
            pltpu.VMEM((B, N, D), jnp.float32),
            pltpu.VMEM((1, C1), jnp.float32),
            pltpu.VMEM((1, C1), jnp.float32),
            pltpu.VMEM((1, D), jnp.float32),
            pltpu.VMEM((1, D), jnp.float32),
        ],
    )(g4, u, v, gamma1, beta1, W2, gamma2, beta2)


def kernel(x, pos, W1, gamma1, beta1, W2, gamma2, beta2):
    u, v, idx = _stageA(x, pos, W1)
    g = _sc_gather(u.reshape(B * N, C1), idx.reshape(NG))
    g4 = g.reshape(B, K - 1, N, C1)
    return _stage4(g4, u, v, gamma1.reshape(1, C1), beta1.reshape(1, C1),
                   W2, gamma2.reshape(1, D), beta2.reshape(1, D))
